# baseline (speedup 1.0000x reference)
import functools
import math

import jax
import jax.numpy as jnp
from jax import lax
from jax.experimental import pallas as pl
from jax.experimental.pallas import tpu as pltpu
from jax.experimental.pallas import tpu_sc as plsc

VOCAB = 1000000
EMB = 64
SCALE = math.sqrt(EMB)

_NUM_CORES = 2
_NUM_SUBCORES = 16
_NW = _NUM_CORES * _NUM_SUBCORES

_BATCH = 4096
_SEQ = 200
_HSEQ = _SEQ // 2
_IDXROW = 2 * _HSEQ + 8
_GLEN = _HSEQ + 4
_ODD_OFF = _HSEQ + 4
_B_PER_W = _BATCH // _NW
_NGB = 3
_NPB = 2
_LOOK = 2


def _sc_embed(tokens_arr, table):
    mesh = plsc.VectorSubcoreMesh(
        core_axis_name="c", subcore_axis_name="s")

    @functools.partial(
        pl.kernel,
        out_type=jax.ShapeDtypeStruct((_BATCH, _HSEQ, 2 * EMB), jnp.float32),
        mesh=mesh,
        scratch_types=[
            pltpu.VMEM((_B_PER_W, _IDXROW), jnp.int32),
            [pltpu.VMEM((_GLEN, EMB), jnp.float32)] * _NGB,
            [pltpu.VMEM((_GLEN, EMB), jnp.float32)] * _NGB,
            [pltpu.VMEM((_HSEQ, 2 * EMB), jnp.float32)] * _NPB,
            [pltpu.SemaphoreType.DMA] * _NGB,
            [pltpu.SemaphoreType.DMA] * _NGB,
            [pltpu.SemaphoreType.DMA] * _NPB,
        ],
        compiler_params=pltpu.CompilerParams(use_tc_tiling_on_sc=False),
    )
    def body(tok_hbm, table_hbm, out_hbm, idx_all,
             bufe, bufo, pair, esem, osem, wsem):
        wid = lax.axis_index("s") * _NUM_CORES + lax.axis_index("c")
        base = wid * _B_PER_W

        pltpu.sync_copy(tok_hbm.at[pl.ds(base, _B_PER_W)], idx_all)

        def gathers(g, b):
            return (
                pltpu.make_async_copy(
                    table_hbm.at[idx_all.at[g, pl.ds(0, _GLEN)]],
                    bufe[b], esem[b]),
                pltpu.make_async_copy(
                    table_hbm.at[idx_all.at[g, pl.ds(_ODD_OFF, _GLEN)]],
                    bufo[b], osem[b]),
            )

        def write(g, b):
            return pltpu.make_async_copy(
                pair[b], out_hbm.at[base + g], wsem[b])

        def scale_interleave(gb, pb):
            e, o, p = bufe[gb], bufo[gb], pair[pb]

            @plsc.parallel_loop(0, _HSEQ, unroll=4)
            def _(i):
                for j in range(EMB // 16):
                    sl = pl.ds(j * 16, 16)
                    so = pl.ds(EMB + j * 16, 16)
                    p[i, sl] = e[i, sl] * SCALE
                    p[i, so] = o[i, sl] * SCALE

        def step(g, gb, pb, pfb, wait_write, prefetch):
            ge, go = gathers(g, gb)
            ge.wait()
            go.wait()
            scale_interleave(gb, pb)
            if wait_write:
                write(g - _NPB, pb).wait()
            write(g, pb).start()
            if prefetch:
                fe, fo = gathers(g + _LOOK, pfb)
                fe.start()
                fo.start()

        for j in range(_LOOK):
            ge, go = gathers(j, j)
            ge.start()
            go.start()
        for g in range(_LOOK):
            step(g, g % _NGB, g % _NPB, (g + _LOOK) % _NGB,
                 wait_write=False, prefetch=True)
        nblocks = (_B_PER_W - _LOOK - 4) // 6

        def block(G, carry):
            for b in range(6):
                g = _LOOK + G * 6 + b
                step(g, (_LOOK + b) % _NGB, b % _NPB,
                     (2 * _LOOK + b) % _NGB,
                     wait_write=True, prefetch=True)
            return carry

        lax.fori_loop(0, nblocks, block, 0)
        for g in range(_B_PER_W - 6, _B_PER_W - _LOOK):
            step(g, g % _NGB, g % _NPB, (g + _LOOK) % _NGB,
                 wait_write=True, prefetch=True)
        for g in range(_B_PER_W - _LOOK, _B_PER_W):
            step(g, g % _NGB, g % _NPB, 0,
                 wait_write=True, prefetch=False)
        for g in range(_B_PER_W - _NPB, _B_PER_W):
            write(g, g % _NPB).wait()

    return body(tokens_arr, table)


def kernel(tokens, table):
    tok = tokens.astype(jnp.int32)
    pad = jnp.zeros((_BATCH, 4), jnp.int32)
    arranged = jnp.concatenate(
        [tok[:, 0::2], pad, tok[:, 1::2], pad], axis=1)
    out = _sc_embed(arranged, table)
    return out.reshape(_BATCH, _SEQ, EMB)


# speedup vs baseline: 1.5983x; 1.5983x over previous
import functools
import math

import jax
import jax.numpy as jnp
from jax import lax
from jax.experimental import pallas as pl
from jax.experimental.pallas import tpu as pltpu
from jax.experimental.pallas import tpu_sc as plsc

VOCAB = 1000000
EMB = 64
SCALE = math.sqrt(EMB)

_NUM_CORES = 2
_NUM_SUBCORES = 16
_NW = _NUM_CORES * _NUM_SUBCORES

_BATCH = 4096
_SEQ = 200
_B_PER_W = _BATCH // _NW
_NBUF = 4
_LOOKAHEAD = 2


def _sc_embed(tokens, table):
    mesh = plsc.VectorSubcoreMesh(
        core_axis_name="c", subcore_axis_name="s")

    @functools.partial(
        pl.kernel,
        out_type=jax.ShapeDtypeStruct((_BATCH, _SEQ, EMB), jnp.float32),
        mesh=mesh,
        scratch_types=[
            pltpu.VMEM((_B_PER_W, _SEQ), jnp.int32),
            [pltpu.VMEM((_SEQ, EMB), jnp.float32)] * _NBUF,
            [pltpu.SemaphoreType.DMA] * _NBUF,
            [pltpu.SemaphoreType.DMA] * _NBUF,
        ],
        compiler_params=pltpu.CompilerParams(use_tc_tiling_on_sc=False),
    )
    def body(tok_hbm, table_hbm, out_hbm, idx_all, rows, gsem, wsem):
        wid = lax.axis_index("s") * _NUM_CORES + lax.axis_index("c")
        base = wid * _B_PER_W

        pltpu.sync_copy(tok_hbm.at[pl.ds(base, _B_PER_W)], idx_all)

        def gather(g, b):
            return pltpu.make_async_copy(
                table_hbm.at[idx_all.at[g]], rows[b], gsem[b])

        def write(g, b):
            return pltpu.make_async_copy(
                rows[b], out_hbm.at[base + g], wsem[b])

        def scale(b):
            r = rows[b]

            @plsc.parallel_loop(0, _SEQ, unroll=8)
            def _(i):
                for j in range(EMB // 16):
                    sl = pl.ds(j * 16, 16)
                    r[i, sl] = r[i, sl] * SCALE

        def step(g, p, wait_write, prefetch):
            gather(g, p).wait()
            scale(p)
            write(g, p).start()
            if prefetch:
                f = g + _LOOKAHEAD
                q = (p + _LOOKAHEAD) % _NBUF
                if wait_write:
                    write(f - _NBUF, q).wait()
                gather(f, q).start()

        for j in range(_LOOKAHEAD):
            gather(j, j).start()
        for g in range(_NBUF - _LOOKAHEAD):
            step(g, g % _NBUF, wait_write=False, prefetch=True)
        head = _NBUF - _LOOKAHEAD
        nblocks = (_B_PER_W - _NBUF) // _NBUF

        def block(G, carry):
            for b in range(_NBUF):
                g = head + G * _NBUF + b
                step(g, (head + b) % _NBUF, wait_write=True, prefetch=True)
            return carry

        lax.fori_loop(0, nblocks, block, 0)
        for g in range(_B_PER_W - _LOOKAHEAD, _B_PER_W):
            step(g, g % _NBUF, wait_write=False, prefetch=False)
        for b in range(_NBUF):
            g = _B_PER_W - _NBUF + b
            write(g, g % _NBUF).wait()

    return body(tokens, table)


def kernel(tokens, table):
    return _sc_embed(tokens.astype(jnp.int32), table)
